# baseline (device time: 175585 ns/iter reference)
import jax
import jax.numpy as jnp
from jax import lax
from jax.experimental import pallas as pl
from jax.experimental.pallas import tpu as pltpu

N_DEV = 4
E_LOC = 4
N_EXP = 16
CAP = 409


def _neighbor_barrier(left, right):
    barrier = pltpu.get_barrier_semaphore()
    for nbr in (left, right):
        pl.semaphore_signal(
            barrier, inc=1, device_id=(nbr,),
            device_id_type=pl.DeviceIdType.MESH,
        )
    pl.semaphore_wait(barrier, 2)


def _counts_body(c_ref, out_ref, send_sems, recv_sems):
    my = lax.axis_index("i")
    left = (my - 1) % N_DEV
    right = (my + 1) % N_DEV
    _neighbor_barrier(left, right)

    out_ref[0:1, :] = c_ref[...]
    for h in range(N_DEV - 1):
        rdma = pltpu.make_async_remote_copy(
            src_ref=out_ref.at[h:h + 1],
            dst_ref=out_ref.at[h + 1:h + 2],
            send_sem=send_sems.at[h],
            recv_sem=recv_sems.at[h],
            device_id=(right,),
            device_id_type=pl.DeviceIdType.MESH,
        )
        rdma.start()
        rdma.wait()


def _counts_allgather(counts_row):
    return pl.pallas_call(
        _counts_body,
        out_shape=jax.ShapeDtypeStruct((N_DEV, 128), jnp.int32),
        in_specs=[pl.BlockSpec(memory_space=pltpu.VMEM)],
        out_specs=pl.BlockSpec(memory_space=pltpu.VMEM),
        scratch_shapes=[
            pltpu.SemaphoreType.DMA((N_DEV - 1,)),
            pltpu.SemaphoreType.DMA((N_DEV - 1,)),
        ],
        compiler_params=pltpu.CompilerParams(collective_id=1),
    )(counts_row)


def _moe_body(x_ref, w_ref, g_ref, out_ref, wbuf, send_sems, recv_sems):
    my = lax.axis_index("i")
    left = (my - 1) % N_DEV
    right = (my + 1) % N_DEV
    _neighbor_barrier(left, right)

    rdma = pltpu.make_async_remote_copy(
        src_ref=w_ref,
        dst_ref=wbuf.at[0],
        send_sem=send_sems.at[0],
        recv_sem=recv_sems.at[0],
        device_id=(right,),
        device_id_type=pl.DeviceIdType.MESH,
    )
    rdma.start()

    x = x_ref[...]

    def chunk_out(load_w, slot):
        acc = None
        for le in range(E_LOC):
            g = g_ref[:, slot * E_LOC + le:slot * E_LOC + le + 1]
            part = jnp.dot(x * g, load_w(le),
                           preferred_element_type=jnp.float32)
            acc = part if acc is None else acc + part
        return acc

    out_ref[...] = chunk_out(lambda le: w_ref[le], 0)

    for h in range(N_DEV - 1):
        rdma.wait()
        if h + 1 < N_DEV - 1:
            rdma = pltpu.make_async_remote_copy(
                src_ref=wbuf.at[h],
                dst_ref=wbuf.at[h + 1],
                send_sem=send_sems.at[h + 1],
                recv_sem=recv_sems.at[h + 1],
                device_id=(right,),
                device_id_type=pl.DeviceIdType.MESH,
            )
            rdma.start()
        out_ref[...] += chunk_out(lambda le: wbuf[h, le], h + 1)


def _moe_call(x_bf, w_bf, gcols, n_tok, d, hdim):
    return pl.pallas_call(
        _moe_body,
        out_shape=jax.ShapeDtypeStruct((n_tok, hdim), jnp.float32),
        in_specs=[
            pl.BlockSpec(memory_space=pltpu.VMEM),
            pl.BlockSpec(memory_space=pltpu.VMEM),
            pl.BlockSpec(memory_space=pltpu.VMEM),
        ],
        out_specs=pl.BlockSpec(memory_space=pltpu.VMEM),
        scratch_shapes=[
            pltpu.VMEM((N_DEV - 1, E_LOC, d, hdim), jnp.bfloat16),
            pltpu.SemaphoreType.DMA((N_DEV - 1,)),
            pltpu.SemaphoreType.DMA((N_DEV - 1,)),
        ],
        compiler_params=pltpu.CompilerParams(collective_id=0),
    )(x_bf, w_bf, gcols)


def kernel(x, router_W, route_idx, expert_W):
    del router_W
    n_tok, d = x.shape
    hdim = expert_W.shape[-1]
    p = lax.axis_index("i")

    route = route_idx[:, 0]
    oh = route[:, None] == jnp.arange(N_EXP, dtype=route.dtype)[None, :]
    oh_i = oh.astype(jnp.int32)
    lrank = jnp.cumsum(oh_i, axis=0) - oh_i
    counts = oh_i.sum(axis=0)
    counts_row = jnp.zeros((1, 128), jnp.int32).at[0, :N_EXP].set(counts)

    counts_slots = _counts_allgather(counts_row)

    origins = (p - jnp.arange(N_DEV)) % N_DEV
    mask = (origins < p).astype(jnp.int32)
    prefix = (counts_slots[:, :N_EXP] * mask[:, None]).sum(axis=0)

    keep = oh & ((lrank + prefix[None, :]) < CAP)
    gate = keep.astype(jnp.bfloat16)
    cols = (origins[:, None] * E_LOC + jnp.arange(E_LOC)[None, :]).reshape(-1)
    gcols = jnp.take(gate, cols, axis=1)

    x_bf = x.astype(jnp.bfloat16)
    w_bf = expert_W.astype(jnp.bfloat16)
    return _moe_call(x_bf, w_bf, gcols, n_tok, d, hdim)


# device time: 105457 ns/iter; 1.6650x vs baseline; 1.6650x over previous
import jax
import jax.numpy as jnp
from jax import lax
from jax.experimental import pallas as pl
from jax.experimental.pallas import tpu as pltpu

N_DEV = 4
E_LOC = 4
N_EXP = 16
CAP = 409


def _neighbor_barrier(left, right):
    barrier = pltpu.get_barrier_semaphore()
    for nbr in (left, right):
        pl.semaphore_signal(
            barrier, inc=1, device_id=(nbr,),
            device_id_type=pl.DeviceIdType.MESH,
        )
    pl.semaphore_wait(barrier, 2)


def _counts_body(c_ref, out_ref, send_sems, recv_sems):
    my = lax.axis_index("i")
    left = (my - 1) % N_DEV
    right = (my + 1) % N_DEV
    _neighbor_barrier(left, right)

    out_ref[0:1, :] = c_ref[...]
    for h in range(N_DEV - 1):
        rdma = pltpu.make_async_remote_copy(
            src_ref=out_ref.at[h:h + 1],
            dst_ref=out_ref.at[h + 1:h + 2],
            send_sem=send_sems.at[h],
            recv_sem=recv_sems.at[h],
            device_id=(right,),
            device_id_type=pl.DeviceIdType.MESH,
        )
        rdma.start()
        rdma.wait()


def _counts_allgather(counts_row):
    return pl.pallas_call(
        _counts_body,
        out_shape=jax.ShapeDtypeStruct((N_DEV, 128), jnp.int32),
        in_specs=[pl.BlockSpec(memory_space=pltpu.VMEM)],
        out_specs=pl.BlockSpec(memory_space=pltpu.VMEM),
        scratch_shapes=[
            pltpu.SemaphoreType.DMA((N_DEV - 1,)),
            pltpu.SemaphoreType.DMA((N_DEV - 1,)),
        ],
        compiler_params=pltpu.CompilerParams(collective_id=1),
    )(counts_row)


def _moe_body(x_ref, w_ref, g_ref, out_ref, wbuf, send_sems, recv_sems):
    my = lax.axis_index("i")
    left = (my - 1) % N_DEV
    right = (my + 1) % N_DEV
    _neighbor_barrier(left, right)

    def msg(k, src, dst, dev):
        return pltpu.make_async_remote_copy(
            src_ref=src, dst_ref=dst,
            send_sem=send_sems.at[k], recv_sem=recv_sems.at[k],
            device_id=(dev,), device_id_type=pl.DeviceIdType.MESH,
        )

    m0 = msg(0, w_ref.at[0:2], wbuf.at[0, 0:2], right)
    m1 = msg(1, w_ref.at[2:4], wbuf.at[0, 2:4], right)
    m2 = msg(2, w_ref.at[0:2], wbuf.at[1, 0:2], left)
    m3 = msg(3, w_ref.at[2:4], wbuf.at[1, 2:4], left)
    m4 = msg(4, wbuf.at[0, 0:2], wbuf.at[2, 0:2], right)
    m5 = msg(5, wbuf.at[1, 2:4], wbuf.at[2, 2:4], left)

    m0.start()
    m1.start()
    m3.start()
    m2.start()

    x = x_ref[...]

    def chunk_out(load_w, slot):
        acc = None
        for le in range(E_LOC):
            g = g_ref[:, slot * E_LOC + le:slot * E_LOC + le + 1]
            part = jnp.dot(x * g, load_w(le),
                           preferred_element_type=jnp.float32)
            acc = part if acc is None else acc + part
        return acc

    out_ref[...] = chunk_out(lambda le: w_ref[le], 0)

    m0.wait_recv()
    m4.start()
    m3.wait_recv()
    m5.start()

    m1.wait_recv()
    out_ref[...] += chunk_out(lambda le: wbuf[0, le], 1)
    m2.wait_recv()
    out_ref[...] += chunk_out(lambda le: wbuf[1, le], 2)
    m4.wait_recv()
    m5.wait_recv()
    out_ref[...] += chunk_out(lambda le: wbuf[2, le], 3)

    for m in (m0, m1, m2, m3, m4, m5):
        m.wait_send()


def _moe_call(x_bf, w_bf, gcols, n_tok, d, hdim):
    return pl.pallas_call(
        _moe_body,
        out_shape=jax.ShapeDtypeStruct((n_tok, hdim), jnp.float32),
        in_specs=[
            pl.BlockSpec(memory_space=pltpu.VMEM),
            pl.BlockSpec(memory_space=pltpu.VMEM),
            pl.BlockSpec(memory_space=pltpu.VMEM),
        ],
        out_specs=pl.BlockSpec(memory_space=pltpu.VMEM),
        scratch_shapes=[
            pltpu.VMEM((N_DEV - 1, E_LOC, d, hdim), jnp.bfloat16),
            pltpu.SemaphoreType.DMA((6,)),
            pltpu.SemaphoreType.DMA((6,)),
        ],
        compiler_params=pltpu.CompilerParams(collective_id=0),
    )(x_bf, w_bf, gcols)


def kernel(x, router_W, route_idx, expert_W):
    del router_W
    n_tok, d = x.shape
    hdim = expert_W.shape[-1]
    p = lax.axis_index("i")

    route = route_idx[:, 0]
    oh = route[:, None] == jnp.arange(N_EXP, dtype=route.dtype)[None, :]
    oh_i = oh.astype(jnp.int32)
    lrank = jnp.cumsum(oh_i, axis=0) - oh_i
    counts = oh_i.sum(axis=0)
    counts_row = jnp.zeros((1, 128), jnp.int32).at[0, :N_EXP].set(counts)

    counts_slots = _counts_allgather(counts_row)

    ring_origins = (p - jnp.arange(N_DEV)) % N_DEV
    mask = (ring_origins < p).astype(jnp.int32)
    prefix = (counts_slots[:, :N_EXP] * mask[:, None]).sum(axis=0)

    keep = oh & ((lrank + prefix[None, :]) < CAP)
    gate = keep.astype(jnp.bfloat16)
    origins = (p + jnp.array([0, -1, 1, 2], jnp.int32)) % N_DEV
    cols = (origins[:, None] * E_LOC + jnp.arange(E_LOC)[None, :]).reshape(-1)
    gcols = jnp.take(gate, cols, axis=1)

    x_bf = x.astype(jnp.bfloat16)
    w_bf = expert_W.astype(jnp.bfloat16)
    return _moe_call(x_bf, w_bf, gcols, n_tok, d, hdim)


# device time: 100095 ns/iter; 1.7542x vs baseline; 1.0536x over previous
import jax
import jax.numpy as jnp
from jax import lax
from jax.experimental import pallas as pl
from jax.experimental.pallas import tpu as pltpu

N_DEV = 4
E_LOC = 4
N_EXP = 16
CAP = 409


def _moe_body(x_ref, w_ref, oh_ref, lr_ref, cnt_ref, pm_ref, out_ref,
              wbuf, cbuf, send_sems, recv_sems):
    my = lax.axis_index("i")
    left = (my - 1) % N_DEV
    right = (my + 1) % N_DEV
    opp = (my + 2) % N_DEV

    barrier = pltpu.get_barrier_semaphore()
    for nbr in (left, right, opp):
        pl.semaphore_signal(
            barrier, inc=1, device_id=(nbr,),
            device_id_type=pl.DeviceIdType.MESH,
        )
    pl.semaphore_wait(barrier, N_DEV - 1)

    def msg(k, src, dst, dev):
        return pltpu.make_async_remote_copy(
            src_ref=src, dst_ref=dst,
            send_sem=send_sems.at[k], recv_sem=recv_sems.at[k],
            device_id=(dev,), device_id_type=pl.DeviceIdType.MESH,
        )

    c6 = msg(6, cnt_ref, cbuf.at[1:2], right)
    c7 = msg(7, cnt_ref, cbuf.at[2:3], left)
    c8 = msg(8, cnt_ref, cbuf.at[3:4], opp)
    c6.start()
    c7.start()
    c8.start()

    m0 = msg(0, w_ref.at[0:2], wbuf.at[0, 0:2], right)
    m1 = msg(1, w_ref.at[2:4], wbuf.at[0, 2:4], right)
    m2 = msg(2, w_ref.at[0:2], wbuf.at[1, 0:2], left)
    m3 = msg(3, w_ref.at[2:4], wbuf.at[1, 2:4], left)
    m4 = msg(4, wbuf.at[0, 0:2], wbuf.at[2, 0:2], right)
    m5 = msg(5, wbuf.at[1, 2:4], wbuf.at[2, 2:4], left)
    m0.start()
    m1.start()
    m3.start()
    m2.start()

    c6.wait_recv()
    c7.wait_recv()
    c8.wait_recv()
    pref = jnp.dot(cnt_ref[:, :N_EXP], pm_ref[0],
                   preferred_element_type=jnp.float32)
    for r in range(1, N_DEV):
        pref += jnp.dot(cbuf[r:r + 1, :N_EXP], pm_ref[r],
                        preferred_element_type=jnp.float32)
    gate = oh_ref[...] * ((lr_ref[...] + pref) < CAP).astype(jnp.bfloat16)

    x = x_ref[...]

    def chunk_out(load_w, slot):
        acc = None
        for le in range(E_LOC):
            g = gate[:, slot * E_LOC + le:slot * E_LOC + le + 1]
            part = jnp.dot(x * g, load_w(le),
                           preferred_element_type=jnp.float32)
            acc = part if acc is None else acc + part
        return acc

    out_ref[...] = chunk_out(lambda le: w_ref[le], 0)

    m0.wait_recv()
    m4.start()
    m3.wait_recv()
    m5.start()

    m1.wait_recv()
    out_ref[...] += chunk_out(lambda le: wbuf[0, le], 1)
    m2.wait_recv()
    out_ref[...] += chunk_out(lambda le: wbuf[1, le], 2)
    m4.wait_recv()
    m5.wait_recv()
    out_ref[...] += chunk_out(lambda le: wbuf[2, le], 3)

    for m in (m0, m1, m2, m3, m4, m5, c6, c7, c8):
        m.wait_send()


def _moe_call(x_bf, w_bf, oh_cols, lr_cols, counts_row, pmats, n_tok, d, hdim):
    return pl.pallas_call(
        _moe_body,
        out_shape=jax.ShapeDtypeStruct((n_tok, hdim), jnp.float32),
        in_specs=[pl.BlockSpec(memory_space=pltpu.VMEM)] * 6,
        out_specs=pl.BlockSpec(memory_space=pltpu.VMEM),
        scratch_shapes=[
            pltpu.VMEM((N_DEV - 1, E_LOC, d, hdim), jnp.bfloat16),
            pltpu.VMEM((N_DEV, 128), jnp.float32),
            pltpu.SemaphoreType.DMA((9,)),
            pltpu.SemaphoreType.DMA((9,)),
        ],
        compiler_params=pltpu.CompilerParams(collective_id=0),
    )(x_bf, w_bf, oh_cols, lr_cols, counts_row, pmats)


def kernel(x, router_W, route_idx, expert_W):
    del router_W
    n_tok, d = x.shape
    hdim = expert_W.shape[-1]
    p = lax.axis_index("i")

    route = route_idx[:, 0]
    oh = route[:, None] == jnp.arange(N_EXP, dtype=route.dtype)[None, :]
    oh_i = oh.astype(jnp.int32)
    lrank = jnp.cumsum(oh_i, axis=0) - oh_i
    counts = oh_i.sum(axis=0)
    counts_row = (jnp.zeros((1, 128), jnp.float32)
                  .at[0, :N_EXP].set(counts.astype(jnp.float32)))

    orig = (p + jnp.array([0, -1, 1, 2], jnp.int32)) % N_DEV
    cols = (orig[:, None] * E_LOC
            + jnp.arange(E_LOC, dtype=jnp.int32)[None, :]).reshape(-1)
    e_ids = jnp.arange(N_EXP, dtype=jnp.int32)
    pmats = ((e_ids[None, :, None] == cols[None, None, :])
             & (orig[:, None, None] < p)
             ).astype(jnp.float32)

    oh_cols = jnp.take(oh.astype(jnp.bfloat16), cols, axis=1)
    lr_cols = jnp.take(lrank.astype(jnp.float32), cols, axis=1)

    x_bf = x.astype(jnp.bfloat16)
    w_bf = expert_W.astype(jnp.bfloat16)
    return _moe_call(x_bf, w_bf, oh_cols, lr_cols, counts_row, pmats,
                     n_tok, d, hdim)


# device time: 99246 ns/iter; 1.7692x vs baseline; 1.0086x over previous
import jax
import jax.numpy as jnp
from jax import lax
from jax.experimental import pallas as pl
from jax.experimental.pallas import tpu as pltpu

N_DEV = 4
E_LOC = 4
N_EXP = 16
CAP = 409


def _moe_body(x_ref, w_ref, oh_ref, lr_ref, cnt_ref, pm_ref, out_ref,
              wbuf, cbuf, send_sems, recv_sems):
    my = lax.axis_index("i")
    left = (my - 1) % N_DEV
    right = (my + 1) % N_DEV
    opp = (my + 2) % N_DEV

    barrier = pltpu.get_barrier_semaphore()
    for nbr in (left, right, opp):
        pl.semaphore_signal(
            barrier, inc=1, device_id=(nbr,),
            device_id_type=pl.DeviceIdType.MESH,
        )
    pl.semaphore_wait(barrier, N_DEV - 1)

    def msg(k, src, dst, dev):
        return pltpu.make_async_remote_copy(
            src_ref=src, dst_ref=dst,
            send_sem=send_sems.at[k], recv_sem=recv_sems.at[k],
            device_id=(dev,), device_id_type=pl.DeviceIdType.MESH,
        )

    c6 = msg(6, cnt_ref, cbuf.at[1:2], right)
    c7 = msg(7, cnt_ref, cbuf.at[2:3], left)
    c8 = msg(8, cnt_ref, cbuf.at[3:4], opp)
    c6.start()
    c7.start()
    c8.start()

    m0 = msg(0, w_ref.at[0:2], wbuf.at[0, 0:2], right)
    m1 = msg(1, w_ref.at[2:4], wbuf.at[0, 2:4], right)
    m2 = msg(2, w_ref.at[0:2], wbuf.at[1, 0:2], left)
    m3 = msg(3, w_ref.at[2:4], wbuf.at[1, 2:4], left)
    m4 = msg(4, wbuf.at[0, 0:2], wbuf.at[2, 0:2], right)
    m5 = msg(5, wbuf.at[1, 2:4], wbuf.at[2, 2:4], left)
    m0.start()
    m1.start()
    m3.start()
    m2.start()

    c6.wait_recv()
    c7.wait_recv()
    c8.wait_recv()
    pref = jnp.dot(cnt_ref[:, :N_EXP], pm_ref[0],
                   preferred_element_type=jnp.float32)
    for r in range(1, N_DEV):
        pref += jnp.dot(cbuf[r:r + 1, :N_EXP], pm_ref[r],
                        preferred_element_type=jnp.float32)
    gate = oh_ref[...] * ((lr_ref[...] + pref) < CAP).astype(jnp.bfloat16)

    x = x_ref[...].astype(jnp.bfloat16)
    d = x.shape[1]

    def chunk_out(w_chunk, slot):
        xg = jnp.concatenate(
            [x * gate[:, slot * E_LOC + le:slot * E_LOC + le + 1]
             for le in range(E_LOC)], axis=1)
        wmat = w_chunk.reshape(E_LOC * d, w_chunk.shape[-1])
        return jnp.dot(xg, wmat, preferred_element_type=jnp.float32)

    out_ref[...] = chunk_out(w_ref[...], 0)

    m0.wait_recv()
    m4.start()
    m3.wait_recv()
    m5.start()

    m1.wait_recv()
    out_ref[...] += chunk_out(wbuf[0], 1)
    m2.wait_recv()
    out_ref[...] += chunk_out(wbuf[1], 2)
    m4.wait_recv()
    m5.wait_recv()
    out_ref[...] += chunk_out(wbuf[2], 3)

    for m in (m0, m1, m2, m3, m4, m5, c6, c7, c8):
        m.wait_send()


def _moe_call(x_bf, w_bf, oh_cols, lr_cols, counts_row, pmats, n_tok, d, hdim):
    return pl.pallas_call(
        _moe_body,
        out_shape=jax.ShapeDtypeStruct((n_tok, hdim), jnp.float32),
        in_specs=[pl.BlockSpec(memory_space=pltpu.VMEM)] * 6,
        out_specs=pl.BlockSpec(memory_space=pltpu.VMEM),
        scratch_shapes=[
            pltpu.VMEM((N_DEV - 1, E_LOC, d, hdim), jnp.bfloat16),
            pltpu.VMEM((N_DEV, 128), jnp.float32),
            pltpu.SemaphoreType.DMA((9,)),
            pltpu.SemaphoreType.DMA((9,)),
        ],
        compiler_params=pltpu.CompilerParams(collective_id=0),
    )(x_bf, w_bf, oh_cols, lr_cols, counts_row, pmats)


def kernel(x, router_W, route_idx, expert_W):
    del router_W
    n_tok, d = x.shape
    hdim = expert_W.shape[-1]
    p = lax.axis_index("i")

    route = route_idx[:, 0]
    oh = route[:, None] == jnp.arange(N_EXP, dtype=route.dtype)[None, :]
    oh_i = oh.astype(jnp.int32)
    lrank = jnp.cumsum(oh_i, axis=0) - oh_i
    counts = oh_i.sum(axis=0)
    counts_row = (jnp.zeros((1, 128), jnp.float32)
                  .at[0, :N_EXP].set(counts.astype(jnp.float32)))

    orig = (p + jnp.array([0, -1, 1, 2], jnp.int32)) % N_DEV
    cols = (orig[:, None] * E_LOC
            + jnp.arange(E_LOC, dtype=jnp.int32)[None, :]).reshape(-1)
    e_ids = jnp.arange(N_EXP, dtype=jnp.int32)
    pmats = ((e_ids[None, :, None] == cols[None, None, :])
             & (orig[:, None, None] < p)
             ).astype(jnp.float32)

    oh_cols = jnp.take(oh.astype(jnp.bfloat16), cols, axis=1)
    lr_cols = jnp.take(lrank.astype(jnp.float32), cols, axis=1)

    w_bf = expert_W.astype(jnp.bfloat16)
    return _moe_call(x, w_bf, oh_cols, lr_cols, counts_row, pmats,
                     n_tok, d, hdim)


# device time: 95593 ns/iter; 1.8368x vs baseline; 1.0382x over previous
import jax
import jax.numpy as jnp
from jax import lax
from jax.experimental import pallas as pl
from jax.experimental.pallas import tpu as pltpu

N_DEV = 4
E_LOC = 4
N_EXP = 16
CAP = 409


def _moe_body(x_ref, w_ref, oh_ref, lr_ref, cnt_ref, pm_ref, out_ref,
              wbuf, cbuf, send_sems, recv_sems):
    my = lax.axis_index("i")
    left = (my - 1) % N_DEV
    right = (my + 1) % N_DEV
    opp = (my + 2) % N_DEV

    barrier = pltpu.get_barrier_semaphore()
    for nbr in (left, right, opp):
        pl.semaphore_signal(
            barrier, inc=1, device_id=(nbr,),
            device_id_type=pl.DeviceIdType.MESH,
        )
    pl.semaphore_wait(barrier, N_DEV - 1)

    def msg(k, src, dst, dev):
        return pltpu.make_async_remote_copy(
            src_ref=src, dst_ref=dst,
            send_sem=send_sems.at[k], recv_sem=recv_sems.at[k],
            device_id=(dev,), device_id_type=pl.DeviceIdType.MESH,
        )

    c6 = msg(6, cnt_ref, cbuf.at[1:2], right)
    c7 = msg(7, cnt_ref, cbuf.at[2:3], left)
    c8 = msg(8, cnt_ref, cbuf.at[3:4], opp)
    c6.start()
    c7.start()
    c8.start()

    m0 = msg(0, w_ref.at[0:2], wbuf.at[0, 0:2], right)
    m1 = msg(1, w_ref.at[2:4], wbuf.at[0, 2:4], right)
    m2 = msg(2, w_ref.at[0:2], wbuf.at[1, 0:2], left)
    m3 = msg(3, w_ref.at[2:4], wbuf.at[1, 2:4], left)
    m4 = msg(4, wbuf.at[0, 0:2], wbuf.at[2, 0:2], right)
    m5 = msg(5, wbuf.at[1, 2:4], wbuf.at[2, 2:4], left)
    m0.start()
    m1.start()
    m3.start()
    m2.start()

    c6.wait_recv()
    c7.wait_recv()
    c8.wait_recv()
    pref = jnp.dot(cnt_ref[...], pm_ref[0],
                   preferred_element_type=jnp.float32)
    for r in range(1, N_DEV):
        pref += jnp.dot(cbuf[r:r + 1, :], pm_ref[r],
                        preferred_element_type=jnp.float32)
    gate = oh_ref[...] * ((lr_ref[...] + pref) < CAP).astype(jnp.bfloat16)

    x = x_ref[...].astype(jnp.bfloat16)
    d = x.shape[1]

    def chunk_out(w_chunk, slot):
        xg = jnp.concatenate(
            [x * gate[:, slot * E_LOC + le:slot * E_LOC + le + 1]
             for le in range(E_LOC)], axis=1)
        wmat = w_chunk.reshape(E_LOC * d, w_chunk.shape[-1])
        return jnp.dot(xg, wmat, preferred_element_type=jnp.float32)

    out_ref[...] = chunk_out(w_ref[...], 0)

    m0.wait_recv()
    m4.start()
    m3.wait_recv()
    m5.start()

    m1.wait_recv()
    out_ref[...] += chunk_out(wbuf[0], 1)
    m2.wait_recv()
    out_ref[...] += chunk_out(wbuf[1], 2)
    m4.wait_recv()
    m5.wait_recv()
    out_ref[...] += chunk_out(wbuf[2], 3)

    for m in (m0, m1, m2, m3, m4, m5, c6, c7, c8):
        m.wait_send()


def _moe_call(x_bf, w_bf, oh_cols, lr_cols, counts_row, pmats, n_tok, d, hdim):
    return pl.pallas_call(
        _moe_body,
        out_shape=jax.ShapeDtypeStruct((n_tok, hdim), jnp.float32),
        in_specs=[pl.BlockSpec(memory_space=pltpu.VMEM)] * 6,
        out_specs=pl.BlockSpec(memory_space=pltpu.VMEM),
        scratch_shapes=[
            pltpu.VMEM((N_DEV - 1, E_LOC, d, hdim), jnp.bfloat16),
            pltpu.VMEM((N_DEV, N_EXP), jnp.float32),
            pltpu.SemaphoreType.DMA((9,)),
            pltpu.SemaphoreType.DMA((9,)),
        ],
        compiler_params=pltpu.CompilerParams(collective_id=0),
    )(x_bf, w_bf, oh_cols, lr_cols, counts_row, pmats)


def kernel(x, router_W, route_idx, expert_W):
    del router_W
    n_tok, d = x.shape
    hdim = expert_W.shape[-1]
    p = lax.axis_index("i")

    route = route_idx[:, 0]
    e_ids = jnp.arange(N_EXP, dtype=route.dtype)
    counts = jnp.sum(route[:, None] == e_ids[None, :], axis=0)
    counts_row = counts.astype(jnp.float32).reshape(1, N_EXP)

    orig = (p + jnp.array([0, -1, 1, 2], jnp.int32)) % N_DEV
    cols = (orig[:, None] * E_LOC
            + jnp.arange(E_LOC, dtype=jnp.int32)[None, :]).reshape(-1)
    pmats = ((e_ids[None, :, None] == cols[None, None, :])
             & (orig[:, None, None] < p)
             ).astype(jnp.float32)

    oh_f = (route[:, None] == cols[None, :]).astype(jnp.float32)
    n_blk, blk = 16, n_tok // 16
    a = oh_f.reshape(n_blk, blk, N_EXP)
    tri = jnp.tril(jnp.ones((blk, blk), jnp.float32))
    within = jnp.matmul(tri[None], a)
    bsums = a.sum(axis=1)
    tri_s = jnp.tril(jnp.ones((n_blk, n_blk), jnp.float32), k=-1)
    offs = jnp.matmul(tri_s, bsums)
    lr_cols = (within + offs[:, None, :]).reshape(n_tok, N_EXP) - oh_f
    oh_cols = oh_f.astype(jnp.bfloat16)

    w_bf = expert_W.astype(jnp.bfloat16)
    return _moe_call(x, w_bf, oh_cols, lr_cols, counts_row, pmats,
                     n_tok, d, hdim)
